# baseline (speedup 1.0000x reference)
import functools

import jax
import jax.numpy as jnp
from jax import lax
from jax.experimental import pallas as pl
from jax.experimental.pallas import tpu as pltpu
from jax.experimental.pallas import tpu_sc as plsc

NUM_CORES = 2
NUM_SUBCORES = 16
NUM_WORKERS = NUM_CORES * NUM_SUBCORES
CHUNK = 64
DIM = 64
NBUF = 8


def _make_kernel(n_flat: int):
    mesh = plsc.VectorSubcoreMesh(core_axis_name="c", subcore_axis_name="s")
    blk = n_flat // NUM_WORKERS
    n_chunks = blk // CHUNK

    @functools.partial(
        pl.kernel,
        out_type=jax.ShapeDtypeStruct((n_flat, DIM), jnp.float32),
        mesh=mesh,
        scratch_types=[
            pltpu.VMEM((n_chunks, CHUNK), jnp.int32),
            pltpu.VMEM((NBUF, CHUNK, DIM), jnp.float32),
            pltpu.SemaphoreType.DMA,
            pltpu.SemaphoreType.DMA,
        ],
        compiler_params=pltpu.CompilerParams(use_tc_tiling_on_sc=False),
    )
    def k(x_hbm, w_hbm, out_hbm, idx_v, bufs, gsem, wsem):
        wid = lax.axis_index("s") * NUM_CORES + lax.axis_index("c")
        base = wid * blk
        pltpu.sync_copy(x_hbm.at[pl.ds(base // CHUNK, n_chunks)], idx_v)

        def fire(n):
            pltpu.async_copy(w_hbm.at[idx_v.at[n]], bufs.at[n % NBUF], gsem)

        for n in range(NBUF - 1):
            fire(n)

        @pl.loop(0, n_chunks)
        def _(n):
            s = n % NBUF
            pltpu.make_async_copy(w_hbm.at[pl.ds(0, CHUNK)], bufs.at[s],
                                  gsem).wait()
            pltpu.async_copy(bufs.at[s],
                             out_hbm.at[pl.ds(base + n * CHUNK, CHUNK)], wsem)

            @pl.when(n + NBUF - 1 < n_chunks)
            def _():
                @pl.when(n >= 1)
                def _():
                    pltpu.make_async_copy(bufs.at[0],
                                          out_hbm.at[pl.ds(0, CHUNK)],
                                          wsem).wait()

                fire(n + NBUF - 1)

        for _ in range(NBUF):
            pltpu.make_async_copy(bufs.at[0], out_hbm.at[pl.ds(0, CHUNK)],
                                  wsem).wait()

    return k


def kernel(x, weight):
    rows, cols = x.shape
    n_flat = rows * cols
    x2d = x.reshape(n_flat // CHUNK, CHUNK).astype(jnp.int32)
    out = _make_kernel(n_flat)(x2d, weight)
    return out.reshape(rows, cols, DIM)


# speedup vs baseline: 1.0425x; 1.0425x over previous
import functools

import jax
import jax.numpy as jnp
from jax import lax
from jax.experimental import pallas as pl
from jax.experimental.pallas import tpu as pltpu
from jax.experimental.pallas import tpu_sc as plsc
from jax.experimental import layout as jex_layout

NUM_CORES = 2
NUM_SUBCORES = 16
NUM_WORKERS = NUM_CORES * NUM_SUBCORES
CHUNK = 128
DIM = 64
NBUF = 8


def _make_kernel(n_chunks: int):
    mesh = plsc.VectorSubcoreMesh(core_axis_name="c", subcore_axis_name="s")
    n_blocks = NUM_WORKERS * n_chunks

    @functools.partial(
        pl.kernel,
        out_type=jax.ShapeDtypeStruct((n_blocks, CHUNK, DIM), jnp.float32),
        mesh=mesh,
        scratch_types=[
            pltpu.VMEM((n_chunks, CHUNK), jnp.int32),
            pltpu.VMEM((NBUF, CHUNK, DIM), jnp.float32),
            pltpu.SemaphoreType.DMA,
            pltpu.SemaphoreType.DMA,
        ],
        compiler_params=pltpu.CompilerParams(use_tc_tiling_on_sc=False),
    )
    def k(x_hbm, w_hbm, out_hbm, idx_v, bufs, gsem, wsem):
        wid = lax.axis_index("s") * NUM_CORES + lax.axis_index("c")
        base = wid * n_chunks
        pltpu.sync_copy(x_hbm.at[pl.ds(base, n_chunks)], idx_v)

        for t in range(NBUF - 1):
            pltpu.async_copy(w_hbm.at[idx_v.at[t]], bufs.at[t], gsem)

        @pl.loop(0, n_chunks)
        def _(j):
            s = j % NBUF
            pltpu.make_async_copy(w_hbm.at[pl.ds(0, CHUNK)], bufs.at[s],
                                  gsem).wait()
            pltpu.async_copy(bufs.at[s], out_hbm.at[base + j], wsem)
            nj = j + NBUF - 1

            @pl.when(nj < n_chunks)
            def _():
                @pl.when(j >= 1)
                def _():
                    pltpu.make_async_copy(bufs.at[0], out_hbm.at[0],
                                          wsem).wait()

                pltpu.async_copy(w_hbm.at[idx_v.at[nj]], bufs.at[nj % NBUF],
                                 gsem)

        for _ in range(NBUF):
            pltpu.make_async_copy(bufs.at[0], out_hbm.at[0], wsem).wait()

    return k


def kernel(x, weight):
    rows, cols = x.shape
    b = rows * cols
    n_chunks = b // (NUM_WORKERS * CHUNK)
    x2 = x.T.reshape(b // CHUNK, CHUNK).astype(jnp.int32)
    x2 = jex_layout.with_layout_constraint(
        x2,
        jex_layout.Layout(major_to_minor=(0, 1), tiling=((8, 128),)))
    out = _make_kernel(n_chunks)(x2, weight)
    return out.reshape(cols, rows, DIM).transpose(1, 0, 2)
